# baseline (speedup 1.0000x reference)
import jax
import jax.numpy as jnp
from jax.experimental import pallas as pl
from jax.experimental.pallas import tpu as pltpu

_D_MODEL = 768
_D_EMB = 64
_E = 64
_BASIS = 8
_EB = _E * _BASIS
_TM = 4096
_CHUNKS = 8


def _hrn_block(x_ref, wh_ref, basis_ref, wdec_ref, y_ref, flat_ref):
    f32 = jnp.float32

    @pl.when(pl.program_id(0) == 0)
    def _init():
        basis = basis_ref[...]
        flat_ref[...] = basis * (1.0 / (
            jnp.sqrt(jnp.sum(basis * basis, axis=1, keepdims=True)) + 1e-8))

    wh = wh_ref[...]
    wdec = wdec_ref[...]
    flat = flat_ref[...]

    rows = _TM // _CHUNKS
    R = range(_CHUNKS)

    eTs = [jax.lax.dot_general(wh, x_ref[pl.ds(h * rows, rows), :],
                               (((0,), (1,)), ((), ())),
                               preferred_element_type=f32) for h in R]
    eTs = [eT * (1.0 / (jnp.sqrt(jnp.sum(eT * eT, axis=0, keepdims=True)) + 1e-8))
           for eT in eTs]

    cTs = [jax.lax.dot_general(flat, eT, (((1,), (0,)), ((), ())),
                               preferred_element_type=f32) for eT in eTs]

    sqs = [c * c for c in cTs]
    s4s = [s[0:4 * _E, :] + s[4 * _E:8 * _E, :] for s in sqs]
    s2s = [s[0:2 * _E, :] + s[2 * _E:4 * _E, :] for s in s4s]
    ens = [s[0:_E, :] + s[_E:2 * _E, :] for s in s2s]

    ws = []
    for en in ens:
        m1 = jnp.max(en, axis=0, keepdims=True)
        is1 = en == m1
        en2 = jnp.where(is1, -1.0, en)
        m2 = jnp.max(en2, axis=0, keepdims=True)
        ed = jnp.exp(m2 - m1)
        g1 = 1.0 / (1.0 + ed)
        g2 = ed * g1
        ws.append(jnp.where(is1, g1, jnp.where(en2 == m2, g2, 0.0)))

    cwTs = [jnp.concatenate([c[b * _E:(b + 1) * _E, :] * w for b in range(_BASIS)],
                            axis=0) for c, w in zip(cTs, ws)]
    mixes = [jax.lax.dot_general(cwT, flat, (((0,), (0,)), ((), ())),
                                 preferred_element_type=f32) for cwT in cwTs]

    for h in R:
        y_ref[pl.ds(h * rows, rows), :] = jax.lax.dot_general(
            mixes[h], wdec, (((1,), (0,)), ((), ())), preferred_element_type=f32)


@jax.jit
def kernel(x, W_hash, basis, W_dec):
    t = x.shape[0]
    basis2 = basis.transpose(1, 0, 2).reshape(_EB, _D_EMB)
    return pl.pallas_call(
        _hrn_block,
        grid=(t // _TM,),
        in_specs=[
            pl.BlockSpec((_TM, _D_MODEL), lambda i: (i, 0)),
            pl.BlockSpec((_D_MODEL, _D_EMB), lambda i: (0, 0)),
            pl.BlockSpec((_EB, _D_EMB), lambda i: (0, 0)),
            pl.BlockSpec((_D_EMB, _D_MODEL), lambda i: (0, 0)),
        ],
        out_specs=pl.BlockSpec((_TM, _D_MODEL), lambda i: (i, 0)),
        out_shape=jax.ShapeDtypeStruct((t, _D_MODEL), jnp.float32),
        scratch_shapes=[pltpu.VMEM((_EB, _D_EMB), jnp.float32)],
    )(x, W_hash, basis2, W_dec)


# speedup vs baseline: 1.0067x; 1.0067x over previous
import jax
import jax.numpy as jnp
from jax.experimental import pallas as pl
from jax.experimental.pallas import tpu as pltpu

_D_MODEL = 768
_D_EMB = 64
_E = 64
_BASIS = 8
_EB = _E * _BASIS
_TM = 4096
_CHUNKS = 2


def _hrn_block(x_ref, wh_ref, basis_ref, wdec_ref, y_ref, flat_ref):
    f32 = jnp.float32

    @pl.when(pl.program_id(0) == 0)
    def _init():
        basis = basis_ref[...]
        flat_ref[...] = basis * (1.0 / (
            jnp.sqrt(jnp.sum(basis * basis, axis=1, keepdims=True)) + 1e-8))

    wh = wh_ref[...]
    wdec = wdec_ref[...]
    flat = flat_ref[...]

    rows = _TM // _CHUNKS
    R = range(_CHUNKS)

    eTs = [jax.lax.dot_general(wh, x_ref[pl.ds(h * rows, rows), :],
                               (((0,), (1,)), ((), ())),
                               preferred_element_type=f32) for h in R]
    eTs = [eT * (1.0 / (jnp.sqrt(jnp.sum(eT * eT, axis=0, keepdims=True)) + 1e-8))
           for eT in eTs]

    cTs = [jax.lax.dot_general(flat, eT, (((1,), (0,)), ((), ())),
                               preferred_element_type=f32) for eT in eTs]

    sqs = [c * c for c in cTs]
    s4s = [s[0:4 * _E, :] + s[4 * _E:8 * _E, :] for s in sqs]
    s2s = [s[0:2 * _E, :] + s[2 * _E:4 * _E, :] for s in s4s]
    ens = [s[0:_E, :] + s[_E:2 * _E, :] for s in s2s]

    ws = []
    for en in ens:
        m1 = jnp.max(en, axis=0, keepdims=True)
        is1 = en == m1
        en2 = jnp.where(is1, -1.0, en)
        m2 = jnp.max(en2, axis=0, keepdims=True)
        ed = jnp.exp(m2 - m1)
        g1 = 1.0 / (1.0 + ed)
        g2 = ed * g1
        ws.append(jnp.where(is1, g1, jnp.where(en2 == m2, g2, 0.0)))

    cwTs = [jnp.concatenate([c[b * _E:(b + 1) * _E, :] * w for b in range(_BASIS)],
                            axis=0) for c, w in zip(cTs, ws)]
    mixes = [jax.lax.dot_general(cwT, flat, (((0,), (0,)), ((), ())),
                                 preferred_element_type=f32) for cwT in cwTs]

    for h in R:
        y_ref[pl.ds(h * rows, rows), :] = jax.lax.dot_general(
            mixes[h], wdec, (((1,), (0,)), ((), ())), preferred_element_type=f32)


@jax.jit
def kernel(x, W_hash, basis, W_dec):
    t = x.shape[0]
    basis2 = basis.transpose(1, 0, 2).reshape(_EB, _D_EMB)
    return pl.pallas_call(
        _hrn_block,
        grid=(t // _TM,),
        in_specs=[
            pl.BlockSpec((_TM, _D_MODEL), lambda i: (i, 0)),
            pl.BlockSpec((_D_MODEL, _D_EMB), lambda i: (0, 0)),
            pl.BlockSpec((_EB, _D_EMB), lambda i: (0, 0)),
            pl.BlockSpec((_D_EMB, _D_MODEL), lambda i: (0, 0)),
        ],
        out_specs=pl.BlockSpec((_TM, _D_MODEL), lambda i: (i, 0)),
        out_shape=jax.ShapeDtypeStruct((t, _D_MODEL), jnp.float32),
        scratch_shapes=[pltpu.VMEM((_EB, _D_EMB), jnp.float32)],
    )(x, W_hash, basis2, W_dec)
